# baseline (speedup 1.0000x reference)
import functools

import jax
import jax.numpy as jnp
from jax import lax
from jax.experimental import pallas as pl
from jax.experimental.pallas import tpu as pltpu
from jax.experimental.pallas import tpu_sc as plsc

_NUM_NODES = 1000000
_H_DIM = 16
_BATCH = 16384

_NC = 2
_NS = 16
_NW = _NC * _NS
_B_PER_W = _BATCH // _NW

_GROUP = 16
_N_GROUPS = _B_PER_W // _GROUP
_N_PAR = 3
_TAIL_START = (_NUM_NODES // 128) * 128
_LAST_FULL_BLOCK = _TAIL_START - 128
_TAIL_SLICE = _NUM_NODES - 128


def _build_lookup():
    mesh = plsc.VectorSubcoreMesh(core_axis_name="c", subcore_axis_name="s")

    @functools.partial(
        pl.kernel,
        mesh=mesh,
        compiler_params=pltpu.CompilerParams(needs_layout_passes=False),
        out_type=jax.ShapeDtypeStruct((_H_DIM, _BATCH), jnp.float32),
        scratch_types=[
            pltpu.VMEM((_B_PER_W,), jnp.int32),
            pltpu.VMEM((_N_PAR, _GROUP, _H_DIM, 128), jnp.float32),
            pltpu.VMEM((_H_DIM, 128), jnp.float32),
            pltpu.VMEM((_H_DIM, _B_PER_W), jnp.float32),
            pltpu.SemaphoreType.DMA,
        ],
    )
    def lookup(table_hbm, tail_hbm, idx_hbm, out_hbm,
               idx_v, bufs_v, tail_v, cols_v, sem):
        wid = lax.axis_index("s") * _NC + lax.axis_index("c")
        base = wid * _B_PER_W
        pltpu.sync_copy(idx_hbm.at[pl.ds(base, _B_PER_W)], idx_v)
        pltpu.sync_copy(tail_hbm, tail_v)

        lanes = lax.iota(jnp.int32, _H_DIM)

        def group_vecs(g):
            start = pl.multiple_of(g * _GROUP, _GROUP)
            kv = idx_v[pl.ds(start, _GROUP)]
            blk = jnp.minimum((kv >> 7) << 7, _LAST_FULL_BLOCK)
            return kv, blk

        def fetch_group(g, parity):
            _, blk = group_vecs(g)
            for j in range(_GROUP):
                bj = pl.multiple_of(blk[j], 128)
                pltpu.async_copy(
                    table_hbm.at[:, pl.ds(bj, 128)],
                    bufs_v.at[parity, j],
                    sem,
                )

        def consume_group(g, parity):
            for j in range(_GROUP):
                pltpu.make_async_copy(
                    table_hbm.at[:, pl.ds(0, 128)],
                    bufs_v.at[parity, j],
                    sem,
                ).wait()
            kv, blk = group_vecs(g)
            off_main = jnp.minimum(kv - blk, 127)
            off_tail = jnp.clip(kv - _TAIL_SLICE, 0, 127)
            for j in range(_GROUP):
                col_main = plsc.load_gather(
                    bufs_v.at[parity, j],
                    [lanes, jnp.full((_H_DIM,), off_main[j], jnp.int32)],
                )
                col_tail = plsc.load_gather(
                    tail_v,
                    [lanes, jnp.full((_H_DIM,), off_tail[j], jnp.int32)],
                )
                tv = jnp.full((_H_DIM,), kv[j], jnp.int32) >= _TAIL_START
                col = jnp.where(tv, col_tail, col_main)
                plsc.store_scatter(
                    cols_v,
                    [lanes, jnp.full((_H_DIM,), g * _GROUP + j, jnp.int32)],
                    col,
                )

        fetch_group(0, 0)
        fetch_group(1, 1)

        def body(h, carry):
            g = h * _N_PAR
            fetch_group(g + 2, 2)
            consume_group(g, 0)
            fetch_group(g + 3, 0)
            consume_group(g + 1, 1)
            fetch_group(g + 4, 1)
            consume_group(g + 2, 2)
            return carry

        lax.fori_loop(0, (_N_GROUPS - 2) // _N_PAR, body, 0)
        consume_group(_N_GROUPS - 2, (_N_GROUPS - 2) % _N_PAR)
        consume_group(_N_GROUPS - 1, (_N_GROUPS - 1) % _N_PAR)

        pltpu.sync_copy(cols_v, out_hbm.at[:, pl.ds(base, _B_PER_W)])

    return lookup


_lookup = _build_lookup()


def kernel(g, h, r, norm, table):
    table_t = table.T
    idx = h.reshape(_BATCH)
    tail = lax.slice(table_t, (0, _TAIL_SLICE), (_H_DIM, _NUM_NODES))
    out_t = _lookup(table_t, tail, idx)
    return out_t.T


# speedup vs baseline: 1.0444x; 1.0444x over previous
import functools

import jax
import jax.numpy as jnp
from jax import lax
from jax.experimental import pallas as pl
from jax.experimental.pallas import tpu as pltpu
from jax.experimental.pallas import tpu_sc as plsc

_NUM_NODES = 1000000
_H_DIM = 16
_BATCH = 16384

_NC = 2
_NS = 16
_NW = _NC * _NS
_B_PER_W = _BATCH // _NW

_GROUP = 16
_N_GROUPS = _B_PER_W // _GROUP
_N_PAR = 2
_TAIL_START = (_NUM_NODES // 128) * 128
_LAST_FULL_BLOCK = _TAIL_START - 128
_TAIL_SLICE = _NUM_NODES - 128


def _build_lookup():
    mesh = plsc.VectorSubcoreMesh(core_axis_name="c", subcore_axis_name="s")

    @functools.partial(
        pl.kernel,
        mesh=mesh,
        compiler_params=pltpu.CompilerParams(needs_layout_passes=False),
        out_type=jax.ShapeDtypeStruct((_H_DIM, _BATCH), jnp.float32),
        scratch_types=[
            pltpu.VMEM((_B_PER_W,), jnp.int32),
            pltpu.VMEM((_N_PAR, _GROUP, _H_DIM, 128), jnp.float32),
            pltpu.VMEM((_H_DIM, 128), jnp.float32),
            pltpu.VMEM((_H_DIM, _B_PER_W), jnp.float32),
            pltpu.SemaphoreType.DMA,
        ],
    )
    def lookup(table_hbm, tail_hbm, idx_hbm, out_hbm,
               idx_v, bufs_v, tail_v, cols_v, sem):
        wid = lax.axis_index("s") * _NC + lax.axis_index("c")
        base = wid * _B_PER_W
        pltpu.sync_copy(idx_hbm.at[pl.ds(base, _B_PER_W)], idx_v)
        pltpu.sync_copy(tail_hbm, tail_v)

        lanes = lax.iota(jnp.int32, _H_DIM)

        def group_vecs(g):
            start = pl.multiple_of(g * _GROUP, _GROUP)
            kv = idx_v[pl.ds(start, _GROUP)]
            blk = jnp.minimum((kv >> 7) << 7, _LAST_FULL_BLOCK)
            return kv, blk

        def fetch_group(g, parity):
            _, blk = group_vecs(g)
            for j in range(_GROUP):
                bj = pl.multiple_of(blk[j], 128)
                pltpu.async_copy(
                    table_hbm.at[:, pl.ds(bj, 128)],
                    bufs_v.at[parity, j],
                    sem,
                )

        def consume_group(g, parity):
            kv, blk = group_vecs(g)
            off_main = jnp.minimum(kv - blk, 127)
            off_tail = jnp.clip(kv - _TAIL_SLICE, 0, 127)
            for j in range(_GROUP):
                pltpu.make_async_copy(
                    table_hbm.at[:, pl.ds(0, 128)],
                    bufs_v.at[parity, j],
                    sem,
                ).wait()
                col_main = plsc.load_gather(
                    bufs_v.at[parity, j],
                    [lanes, jnp.full((_H_DIM,), off_main[j], jnp.int32)],
                )
                col_tail = plsc.load_gather(
                    tail_v,
                    [lanes, jnp.full((_H_DIM,), off_tail[j], jnp.int32)],
                )
                tv = jnp.full((_H_DIM,), kv[j], jnp.int32) >= _TAIL_START
                col = jnp.where(tv, col_tail, col_main)
                plsc.store_scatter(
                    cols_v,
                    [lanes, jnp.full((_H_DIM,), g * _GROUP + j, jnp.int32)],
                    col,
                )

        fetch_group(0, 0)

        def body(h, carry):
            g = h * 2
            fetch_group(g + 1, 1)
            consume_group(g, 0)

            @pl.when(g + 2 < _N_GROUPS)
            def _():
                fetch_group(g + 2, 0)

            consume_group(g + 1, 1)
            return carry

        lax.fori_loop(0, _N_GROUPS // 2, body, 0)

        pltpu.sync_copy(cols_v, out_hbm.at[:, pl.ds(base, _B_PER_W)])

    return lookup


_lookup = _build_lookup()


def kernel(g, h, r, norm, table):
    table_t = table.T
    idx = h.reshape(_BATCH)
    tail = lax.slice(table_t, (0, _TAIL_SLICE), (_H_DIM, _NUM_NODES))
    out_t = _lookup(table_t, tail, idx)
    return out_t.T


# speedup vs baseline: 1.0445x; 1.0001x over previous
import functools

import jax
import jax.numpy as jnp
from jax import lax
from jax.experimental import pallas as pl
from jax.experimental.pallas import tpu as pltpu
from jax.experimental.pallas import tpu_sc as plsc

_NUM_NODES = 1000000
_H_DIM = 16
_BATCH = 16384

_NC = 2
_NS = 16
_NW = _NC * _NS
_B_PER_W = _BATCH // _NW

_GROUP = 16
_N_GROUPS = _B_PER_W // _GROUP
_N_PAR = 2
_TAIL_START = (_NUM_NODES // 128) * 128
_LAST_FULL_BLOCK = _TAIL_START - 128
_TAIL_SLICE = _NUM_NODES - 128


def _build_lookup():
    mesh = plsc.VectorSubcoreMesh(core_axis_name="c", subcore_axis_name="s")

    @functools.partial(
        pl.kernel,
        mesh=mesh,
        compiler_params=pltpu.CompilerParams(needs_layout_passes=False),
        out_type=jax.ShapeDtypeStruct((_H_DIM, _BATCH), jnp.float32),
        scratch_types=[
            pltpu.VMEM((_B_PER_W,), jnp.int32),
            pltpu.VMEM((_N_PAR, _GROUP, _H_DIM, 128), jnp.float32),
            pltpu.VMEM((_H_DIM, 128), jnp.float32),
            pltpu.VMEM((_H_DIM, _B_PER_W), jnp.float32),
            pltpu.SemaphoreType.DMA,
        ],
    )
    def lookup(table_hbm, tail_hbm, idx_hbm, out_hbm,
               idx_v, bufs_v, tail_v, cols_v, sem):
        wid = lax.axis_index("s") * _NC + lax.axis_index("c")
        base = wid * _B_PER_W
        pltpu.sync_copy(idx_hbm.at[pl.ds(base, _B_PER_W)], idx_v)
        pltpu.sync_copy(tail_hbm, tail_v)

        lanes = lax.iota(jnp.int32, _H_DIM)

        def group_vecs(g):
            start = pl.multiple_of(g * _GROUP, _GROUP)
            kv = idx_v[pl.ds(start, _GROUP)]
            blk = jnp.minimum((kv >> 7) << 7, _LAST_FULL_BLOCK)
            return kv, blk

        def fetch_group(g, parity):
            _, blk = group_vecs(g)
            for j in range(_GROUP):
                bj = pl.multiple_of(blk[j], 128)
                for half in range(2):
                    pltpu.async_copy(
                        table_hbm.at[
                            pl.ds(half * 8, 8), pl.ds(bj, 128)
                        ],
                        bufs_v.at[parity, j, pl.ds(half * 8, 8)],
                        sem,
                    )

        def consume_group(g, parity):
            kv, blk = group_vecs(g)
            off_main = jnp.minimum(kv - blk, 127)
            off_tail = jnp.clip(kv - _TAIL_SLICE, 0, 127)
            for j in range(_GROUP):
                for half in range(2):
                    pltpu.make_async_copy(
                        table_hbm.at[pl.ds(half * 8, 8), pl.ds(0, 128)],
                        bufs_v.at[parity, j, pl.ds(half * 8, 8)],
                        sem,
                    ).wait()
                col_main = plsc.load_gather(
                    bufs_v.at[parity, j],
                    [lanes, jnp.full((_H_DIM,), off_main[j], jnp.int32)],
                )
                col_tail = plsc.load_gather(
                    tail_v,
                    [lanes, jnp.full((_H_DIM,), off_tail[j], jnp.int32)],
                )
                tv = jnp.full((_H_DIM,), kv[j], jnp.int32) >= _TAIL_START
                col = jnp.where(tv, col_tail, col_main)
                plsc.store_scatter(
                    cols_v,
                    [lanes, jnp.full((_H_DIM,), g * _GROUP + j, jnp.int32)],
                    col,
                )

        fetch_group(0, 0)

        def body(h, carry):
            g = h * 2
            fetch_group(g + 1, 1)
            consume_group(g, 0)

            @pl.when(g + 2 < _N_GROUPS)
            def _():
                fetch_group(g + 2, 0)

            consume_group(g + 1, 1)
            return carry

        lax.fori_loop(0, _N_GROUPS // 2, body, 0)

        pltpu.sync_copy(cols_v, out_hbm.at[:, pl.ds(base, _B_PER_W)])

    return lookup


_lookup = _build_lookup()


def kernel(g, h, r, norm, table):
    table_t = table.T
    idx = h.reshape(_BATCH)
    tail = lax.slice(table_t, (0, _TAIL_SLICE), (_H_DIM, _NUM_NODES))
    out_t = _lookup(table_t, tail, idx)
    return out_t.T


# speedup vs baseline: 1.0467x; 1.0021x over previous
import functools

import jax
import jax.numpy as jnp
from jax import lax
from jax.experimental import pallas as pl
from jax.experimental.pallas import tpu as pltpu
from jax.experimental.pallas import tpu_sc as plsc

_NUM_NODES = 1000000
_H_DIM = 16
_BATCH = 16384

_NC = 2
_NS = 16
_NW = _NC * _NS
_B_PER_W = _BATCH // _NW

_GROUP = 16
_N_GROUPS = _B_PER_W // _GROUP
_N_PAR = 2
_TAIL_START = (_NUM_NODES // 128) * 128
_LAST_FULL_BLOCK = _TAIL_START - 128
_TAIL_SLICE = _NUM_NODES - 128


def _build_lookup():
    mesh = plsc.VectorSubcoreMesh(core_axis_name="c", subcore_axis_name="s")

    @functools.partial(
        pl.kernel,
        mesh=mesh,
        compiler_params=pltpu.CompilerParams(
            needs_layout_passes=False,
            disable_bounds_checks=True,
            disable_semaphore_checks=True,
        ),
        out_type=jax.ShapeDtypeStruct((_H_DIM, _BATCH), jnp.float32),
        scratch_types=[
            pltpu.VMEM((_B_PER_W,), jnp.int32),
            pltpu.VMEM((_N_PAR, _GROUP, _H_DIM, 128), jnp.float32),
            pltpu.VMEM((_H_DIM, 128), jnp.float32),
            pltpu.VMEM((_H_DIM, _B_PER_W), jnp.float32),
            pltpu.SemaphoreType.DMA,
        ],
    )
    def lookup(table_hbm, tail_hbm, idx_hbm, out_hbm,
               idx_v, bufs_v, tail_v, cols_v, sem):
        wid = lax.axis_index("s") * _NC + lax.axis_index("c")
        base = wid * _B_PER_W
        pltpu.sync_copy(idx_hbm.at[pl.ds(base, _B_PER_W)], idx_v)
        pltpu.sync_copy(tail_hbm, tail_v)

        lanes = lax.iota(jnp.int32, _H_DIM)

        def group_vecs(g):
            start = pl.multiple_of(g * _GROUP, _GROUP)
            kv = idx_v[pl.ds(start, _GROUP)]
            blk = jnp.minimum((kv >> 7) << 7, _LAST_FULL_BLOCK)
            return kv, blk

        def fetch_group(g, parity):
            _, blk = group_vecs(g)
            for j in range(_GROUP):
                bj = pl.multiple_of(blk[j], 128)
                pltpu.async_copy(
                    table_hbm.at[:, pl.ds(bj, 128)],
                    bufs_v.at[parity, j],
                    sem,
                )

        def consume_group(g, parity):
            kv, blk = group_vecs(g)
            off_main = jnp.minimum(kv - blk, 127)
            off_tail = jnp.clip(kv - _TAIL_SLICE, 0, 127)
            for j in range(_GROUP):
                pltpu.make_async_copy(
                    table_hbm.at[:, pl.ds(0, 128)],
                    bufs_v.at[parity, j],
                    sem,
                ).wait()
                col_main = plsc.load_gather(
                    bufs_v.at[parity, j],
                    [lanes, jnp.full((_H_DIM,), off_main[j], jnp.int32)],
                )
                col_tail = plsc.load_gather(
                    tail_v,
                    [lanes, jnp.full((_H_DIM,), off_tail[j], jnp.int32)],
                )
                tv = jnp.full((_H_DIM,), kv[j], jnp.int32) >= _TAIL_START
                col = jnp.where(tv, col_tail, col_main)
                plsc.store_scatter(
                    cols_v,
                    [lanes, jnp.full((_H_DIM,), g * _GROUP + j, jnp.int32)],
                    col,
                )

        fetch_group(0, 0)

        def body(h, carry):
            g = h * 2
            fetch_group(g + 1, 1)
            consume_group(g, 0)

            @pl.when(g + 2 < _N_GROUPS)
            def _():
                fetch_group(g + 2, 0)

            consume_group(g + 1, 1)
            return carry

        lax.fori_loop(0, _N_GROUPS // 2, body, 0)

        pltpu.sync_copy(cols_v, out_hbm.at[:, pl.ds(base, _B_PER_W)])

    return lookup


_lookup = _build_lookup()


def kernel(g, h, r, norm, table):
    table_t = table.T
    idx = h.reshape(_BATCH)
    tail = lax.slice(table_t, (0, _TAIL_SLICE), (_H_DIM, _NUM_NODES))
    out_t = _lookup(table_t, tail, idx)
    return out_t.T
